# baseline (speedup 1.0000x reference)
import functools

import jax
import jax.numpy as jnp
from jax import lax
from jax.experimental import pallas as pl
from jax.experimental.pallas import tpu as pltpu
from jax.experimental.pallas import tpu_sc as plsc

_BATCH = 4096
_SEQ = 200
_EMBED = 64
_CLASSES = 50
_LANES = 16

_NC = 2
_NS = 16
_NW = _NC * _NS
_RPW = _BATCH // _NW
_G0 = 128
_G1 = _SEQ - _G0
_VOCAB = 1000000
_PAIRS = _VOCAB // 2

_W = 256
_NBF = _VOCAB // _W
_TAILC = _VOCAB - _NBF * _W
_NI = 123


def _tp_body(tab_hbm, out_hbm, slab_a, slab_b, outb_a, outb_b,
             isem0, isem1, osem0, osem1):
    wid = lax.axis_index("s") * _NC + lax.axis_index("c")

    slabs = (slab_a, slab_b)
    outbs = (outb_a, outb_b)
    isems = (isem0, isem1)
    osems = (osem0, osem1)

    iota = jax.lax.iota(jnp.int32, _LANES)
    diags = [(iota + d) % _LANES for d in range(_LANES)]
    cols = [((iota + d) % 2) * _EMBED + iota for d in range(2)]
    ridx = [iota + g * _LANES for g in range(4)]

    def issue_in(c, slot):
        @pl.when(c < _NBF)
        def _():
            off = pl.multiple_of(c * _W, _W)
            pltpu.async_copy(tab_hbm.at[:, pl.ds(off, _W)],
                             slabs[slot], isems[slot])

    def wait_in(slot):
        pltpu.make_async_copy(tab_hbm.at[:, pl.ds(0, _W)],
                              slabs[slot], isems[slot]).wait()

    def transpose(slot, ncb):
        items = [(d, g) for d in range(_LANES) for g in range(4)]
        lookahead = 12

        def cb_body(cb, car):
            cb16 = cb * _LANES
            gidxs = [cb16 + diags[d] for d in range(_LANES)]
            pending = {}
            for t in range(len(items) + lookahead):
                if t < len(items):
                    d, g = items[t]
                    pending[t] = plsc.load_gather(
                        slabs[slot], [ridx[g], gidxs[d]])
                if t >= lookahead:
                    d, g = items[t - lookahead]
                    plsc.store_scatter(
                        outbs[slot],
                        [gidxs[d] >> 1, cols[d % 2] + g * _LANES],
                        pending.pop(t - lookahead))
            return car
        lax.fori_loop(0, ncb, cb_body, 0)

    def issue_out(c, slot):
        off = pl.multiple_of(c * (_W // 2), _W // 2)
        pltpu.async_copy(outbs[slot], out_hbm.at[pl.ds(off, _W // 2)],
                         osems[slot])

    def wait_out_full(slot):
        pltpu.make_async_copy(outbs[slot], out_hbm.at[pl.ds(0, _W // 2)],
                              osems[slot]).wait()

    def step(i, slot, nslot):
        c = wid + 32 * i
        issue_in(c + 32, nslot)

        @pl.when(c < _NBF)
        def _():
            wait_in(slot)

            @pl.when(i >= 2)
            def _():
                wait_out_full(slot)

            transpose(slot, _W // _LANES)
            issue_out(c, slot)

    issue_in(wid, 0)

    def outer(i0, car):
        step(2 * i0, 0, 1)
        step(2 * i0 + 1, 1, 0)
        return car

    lax.fori_loop(0, (_NI - 1) // 2, outer, 0)
    step(_NI - 1, 0, 1)

    wait_out_full(1)
    wait_out_full(0)


_transpose_tab = functools.partial(
    pl.kernel,
    mesh=plsc.VectorSubcoreMesh(core_axis_name="c", subcore_axis_name="s"),
    compiler_params=pltpu.CompilerParams(needs_layout_passes=False),
    out_type=jax.ShapeDtypeStruct((_PAIRS, 2 * _EMBED), jnp.float32),
    scratch_types=[
        pltpu.VMEM((_EMBED, _W), jnp.float32),
        pltpu.VMEM((_EMBED, _W), jnp.float32),
        pltpu.VMEM((_W // 2, 2 * _EMBED), jnp.float32),
        pltpu.VMEM((_W // 2, 2 * _EMBED), jnp.float32),
        pltpu.SemaphoreType.DMA,
        pltpu.SemaphoreType.DMA,
        pltpu.SemaphoreType.DMA,
        pltpu.SemaphoreType.DMA,
    ],
)(_tp_body)


def _pool_body(x2_hbm, xp_hbm, tab_hbm, out_hbm,
               idx_v, par_v, rows_v, pooled_v, sem0, sem1):
    wid = lax.axis_index("s") * _NC + lax.axis_index("c")
    base = pl.multiple_of(wid * _RPW, _RPW)

    pltpu.sync_copy(x2_hbm.at[pl.ds(base * _SEQ, _RPW * _SEQ)], idx_v)
    pltpu.sync_copy(xp_hbm.at[pl.ds(base * _SEQ, _RPW * _SEQ)],
                    par_v.at[pl.ds(0, _RPW * _SEQ)])

    sems = (sem0, sem1)

    def issue(r, slot):
        off = pl.multiple_of(r * _SEQ, 8)
        pltpu.async_copy(tab_hbm.at[idx_v.at[pl.ds(off, _G0)]],
                         rows_v.at[slot, pl.ds(0, _G0)], sems[slot])
        pltpu.async_copy(tab_hbm.at[idx_v.at[pl.ds(off + _G0, _G1)]],
                         rows_v.at[slot, pl.ds(_G0, _G1)], sems[slot])

    def wait(slot):
        pltpu.make_async_copy(tab_hbm.at[idx_v.at[pl.ds(0, _G0)]],
                              rows_v.at[slot, pl.ds(0, _G0)], sems[slot]).wait()
        pltpu.make_async_copy(tab_hbm.at[idx_v.at[pl.ds(0, _G1)]],
                              rows_v.at[slot, pl.ds(_G0, _G1)], sems[slot]).wait()

    issue(0, 0)

    inv = jnp.float32(1.0 / _SEQ)

    def step(r, slot, nslot):
        @pl.when(r + 1 < _RPW)
        def _():
            issue(r + 1, nslot)

        wait(slot)

        zero = jnp.zeros((_LANES,), jnp.float32)

        def body(j, accs):
            pv = par_v[pl.ds(r * _SEQ + j, _LANES)]
            half = pv[0] * _EMBED
            return tuple(
                accs[d] + rows_v[slot, j, pl.ds(half + d * _LANES, _LANES)]
                for d in range(_EMBED // _LANES))

        accs = lax.fori_loop(0, _SEQ, body, (zero,) * (_EMBED // _LANES))
        for d in range(_EMBED // _LANES):
            pooled_v[r, pl.ds(d * _LANES, _LANES)] = accs[d] * inv

    def outer(i, carry):
        step(2 * i, 0, 1)
        step(2 * i + 1, 1, 0)
        return carry

    lax.fori_loop(0, _RPW // 2, outer, 0)

    pltpu.sync_copy(pooled_v, out_hbm.at[pl.ds(base, _RPW)])


_pool = functools.partial(
    pl.kernel,
    mesh=plsc.VectorSubcoreMesh(core_axis_name="c", subcore_axis_name="s"),
    out_type=jax.ShapeDtypeStruct((_BATCH, _EMBED), jnp.float32),
    scratch_types=[
        pltpu.VMEM((_RPW * _SEQ,), jnp.int32),
        pltpu.VMEM((_RPW * _SEQ + _LANES,), jnp.int32),
        pltpu.VMEM((2, _SEQ, 2 * _EMBED), jnp.float32),
        pltpu.VMEM((_RPW, _EMBED), jnp.float32),
        pltpu.SemaphoreType.DMA,
        pltpu.SemaphoreType.DMA,
    ],
)(_pool_body)


def _fc_body(p_ref, w_ref, b_ref, o_ref):
    o_ref[...] = (
        jnp.dot(p_ref[...], w_ref[...], preferred_element_type=jnp.float32)
        + b_ref[...]
    )


def kernel(x, emb_table, fc_w, fc_b):
    tab2 = _transpose_tab(emb_table.T)
    tail = emb_table[_NBF * _W:].reshape(_TAILC // 2, 2 * _EMBED)
    tab2 = jax.lax.dynamic_update_slice(tab2, tail, (_NBF * (_W // 2), 0))
    x2 = jax.lax.shift_right_logical(x, 1).reshape(-1)
    xp = jax.lax.bitwise_and(x, 1).reshape(-1)
    pooled = _pool(x2, xp, tab2)
    return pl.pallas_call(
        _fc_body,
        out_shape=jax.ShapeDtypeStruct((_BATCH, _CLASSES), jnp.float32),
    )(pooled, fc_w, fc_b.reshape(1, _CLASSES))


# speedup vs baseline: 1.0163x; 1.0163x over previous
import functools

import jax
import jax.numpy as jnp
from jax import lax
from jax.experimental import pallas as pl
from jax.experimental.pallas import tpu as pltpu
from jax.experimental.pallas import tpu_sc as plsc

_BATCH = 4096
_SEQ = 200
_EMBED = 64
_CLASSES = 50
_LANES = 16

_NC = 2
_NS = 16
_NW = _NC * _NS
_RPW = _BATCH // _NW
_G0 = 128
_G1 = _SEQ - _G0
_VOCAB = 1000000
_PAIRS = _VOCAB // 2

_W = 256
_NBF = _VOCAB // _W
_TAILC = _VOCAB - _NBF * _W
_NI = 123


def _tp_body(tab_hbm, out_hbm, slab_a, slab_b, outb_a, outb_b,
             isem0, isem1, osem0, osem1):
    wid = lax.axis_index("s") * _NC + lax.axis_index("c")

    slabs = (slab_a, slab_b)
    outbs = (outb_a, outb_b)
    isems = (isem0, isem1)
    osems = (osem0, osem1)

    iota = jax.lax.iota(jnp.int32, _LANES)
    diags = [(iota + d) % _LANES for d in range(_LANES)]
    cols = [((iota + d) % 2) * _EMBED + iota for d in range(2)]
    ridx = [iota + g * _LANES for g in range(4)]

    def issue_in(c, slot):
        @pl.when(c < _NBF)
        def _():
            off = pl.multiple_of(c * _W, _W)
            pltpu.async_copy(tab_hbm.at[:, pl.ds(off, _W)],
                             slabs[slot], isems[slot])

    def wait_in(slot):
        pltpu.make_async_copy(tab_hbm.at[:, pl.ds(0, _W)],
                              slabs[slot], isems[slot]).wait()

    def transpose(slot, ncb):
        items = [(d, g) for d in range(_LANES) for g in range(4)]
        lookahead = 8

        def cb_body(cb, car):
            cb16 = cb * _LANES
            gidxs = [cb16 + diags[d] for d in range(_LANES)]
            pending = {}
            for t in range(len(items) + lookahead):
                if t < len(items):
                    d, g = items[t]
                    pending[t] = plsc.load_gather(
                        slabs[slot], [ridx[g], gidxs[d]])
                if t >= lookahead:
                    d, g = items[t - lookahead]
                    plsc.store_scatter(
                        outbs[slot],
                        [gidxs[d] >> 1, cols[d % 2] + g * _LANES],
                        pending.pop(t - lookahead))
            return car
        lax.fori_loop(0, ncb, cb_body, 0)

    def issue_out(c, slot):
        off = pl.multiple_of(c * (_W // 2), _W // 2)
        pltpu.async_copy(outbs[slot], out_hbm.at[pl.ds(off, _W // 2)],
                         osems[slot])

    def wait_out_full(slot):
        pltpu.make_async_copy(outbs[slot], out_hbm.at[pl.ds(0, _W // 2)],
                              osems[slot]).wait()

    def step(i, slot, nslot):
        c = wid + 32 * i
        issue_in(c + 32, nslot)

        @pl.when(c < _NBF)
        def _():
            wait_in(slot)

            @pl.when(i >= 2)
            def _():
                wait_out_full(slot)

            transpose(slot, _W // _LANES)
            issue_out(c, slot)

    issue_in(wid, 0)

    def outer(i0, car):
        step(2 * i0, 0, 1)
        step(2 * i0 + 1, 1, 0)
        return car

    lax.fori_loop(0, (_NI - 1) // 2, outer, 0)
    step(_NI - 1, 0, 1)

    wait_out_full(1)
    wait_out_full(0)


_transpose_tab = functools.partial(
    pl.kernel,
    mesh=plsc.VectorSubcoreMesh(core_axis_name="c", subcore_axis_name="s"),
    compiler_params=pltpu.CompilerParams(needs_layout_passes=False),
    out_type=jax.ShapeDtypeStruct((_PAIRS, 2 * _EMBED), jnp.float32),
    scratch_types=[
        pltpu.VMEM((_EMBED, _W), jnp.float32),
        pltpu.VMEM((_EMBED, _W), jnp.float32),
        pltpu.VMEM((_W // 2, 2 * _EMBED), jnp.float32),
        pltpu.VMEM((_W // 2, 2 * _EMBED), jnp.float32),
        pltpu.SemaphoreType.DMA,
        pltpu.SemaphoreType.DMA,
        pltpu.SemaphoreType.DMA,
        pltpu.SemaphoreType.DMA,
    ],
)(_tp_body)


def _pool_body(x2_hbm, xp_hbm, tab_hbm, out_hbm,
               idx_v, par_v, rows_v, pooled_v, sem0, sem1):
    wid = lax.axis_index("s") * _NC + lax.axis_index("c")
    base = pl.multiple_of(wid * _RPW, _RPW)

    pltpu.sync_copy(x2_hbm.at[pl.ds(base * _SEQ, _RPW * _SEQ)], idx_v)
    pltpu.sync_copy(xp_hbm.at[pl.ds(base * _SEQ, _RPW * _SEQ)],
                    par_v.at[pl.ds(0, _RPW * _SEQ)])

    sems = (sem0, sem1)

    def issue(r, slot):
        off = pl.multiple_of(r * _SEQ, 8)
        pltpu.async_copy(tab_hbm.at[idx_v.at[pl.ds(off, _G0)]],
                         rows_v.at[slot, pl.ds(0, _G0)], sems[slot])
        pltpu.async_copy(tab_hbm.at[idx_v.at[pl.ds(off + _G0, _G1)]],
                         rows_v.at[slot, pl.ds(_G0, _G1)], sems[slot])

    def wait(slot):
        pltpu.make_async_copy(tab_hbm.at[idx_v.at[pl.ds(0, _G0)]],
                              rows_v.at[slot, pl.ds(0, _G0)], sems[slot]).wait()
        pltpu.make_async_copy(tab_hbm.at[idx_v.at[pl.ds(0, _G1)]],
                              rows_v.at[slot, pl.ds(_G0, _G1)], sems[slot]).wait()

    issue(0, 0)

    inv = jnp.float32(1.0 / _SEQ)

    def step(r, slot, nslot):
        @pl.when(r + 1 < _RPW)
        def _():
            issue(r + 1, nslot)

        wait(slot)

        zero = jnp.zeros((_LANES,), jnp.float32)

        def body(j, accs):
            pv = par_v[pl.ds(r * _SEQ + j, _LANES)]
            half = pv[0] * _EMBED
            return tuple(
                accs[d] + rows_v[slot, j, pl.ds(half + d * _LANES, _LANES)]
                for d in range(_EMBED // _LANES))

        accs = lax.fori_loop(0, _SEQ, body, (zero,) * (_EMBED // _LANES))
        for d in range(_EMBED // _LANES):
            pooled_v[r, pl.ds(d * _LANES, _LANES)] = accs[d] * inv

    def outer(i, carry):
        step(2 * i, 0, 1)
        step(2 * i + 1, 1, 0)
        return carry

    lax.fori_loop(0, _RPW // 2, outer, 0)

    pltpu.sync_copy(pooled_v, out_hbm.at[pl.ds(base, _RPW)])


_pool = functools.partial(
    pl.kernel,
    mesh=plsc.VectorSubcoreMesh(core_axis_name="c", subcore_axis_name="s"),
    out_type=jax.ShapeDtypeStruct((_BATCH, _EMBED), jnp.float32),
    scratch_types=[
        pltpu.VMEM((_RPW * _SEQ,), jnp.int32),
        pltpu.VMEM((_RPW * _SEQ + _LANES,), jnp.int32),
        pltpu.VMEM((2, _SEQ, 2 * _EMBED), jnp.float32),
        pltpu.VMEM((_RPW, _EMBED), jnp.float32),
        pltpu.SemaphoreType.DMA,
        pltpu.SemaphoreType.DMA,
    ],
)(_pool_body)


def _fc_body(p_ref, w_ref, b_ref, o_ref):
    o_ref[...] = (
        jnp.dot(p_ref[...], w_ref[...], preferred_element_type=jnp.float32)
        + b_ref[...]
    )


def kernel(x, emb_table, fc_w, fc_b):
    tab2 = _transpose_tab(emb_table.T)
    tail = emb_table[_NBF * _W:].reshape(_TAILC // 2, 2 * _EMBED)
    tab2 = jax.lax.dynamic_update_slice(tab2, tail, (_NBF * (_W // 2), 0))
    x2 = jax.lax.shift_right_logical(x, 1).reshape(-1)
    xp = jax.lax.bitwise_and(x, 1).reshape(-1)
    pooled = _pool(x2, xp, tab2)
    return pl.pallas_call(
        _fc_body,
        out_shape=jax.ShapeDtypeStruct((_BATCH, _CLASSES), jnp.float32),
    )(pooled, fc_w, fc_b.reshape(1, _CLASSES))
